# baseline (speedup 1.0000x reference)
import jax
import jax.numpy as jnp
from jax.experimental import pallas as pl
from jax.experimental.pallas import tpu as pltpu

_K = 3
_BETA = 0.75
_HIGH = jax.lax.Precision.HIGHEST


def _proj_kernel(res2_ref, wf_ref, proj_ref):
    proj_ref[0] = jax.lax.dot_general(
        wf_ref[...], res2_ref[0], (((1,), (0,)), ((), ())),
        preferred_element_type=jnp.float32, precision=_HIGH)


def _corners(px, py, H, W):
    gx = 2.0 * px - 1.0
    gy = 2.0 * py - 1.0
    fx = ((gx + 1.0) * W - 1.0) / 2.0
    fy = ((gy + 1.0) * H - 1.0) / 2.0
    x0 = jnp.floor(fx)
    y0 = jnp.floor(fy)
    x1 = x0 + 1.0
    y1 = y0 + 1.0
    wx1 = fx - x0
    wx0 = 1.0 - wx1
    wy1 = fy - y0
    wy0 = 1.0 - wy1
    out = []
    for xi, yi, wx, wy in ((x0, y0, wx0, wy0), (x1, y0, wx1, wy0),
                           (x0, y1, wx0, wy1), (x1, y1, wx1, wy1)):
        valid = ((xi >= 0) & (xi <= W - 1) & (yi >= 0) & (yi <= H - 1))
        xc = jnp.clip(xi, 0, W - 1).astype(jnp.int32)
        yc = jnp.clip(yi, 0, H - 1).astype(jnp.int32)
        out.append((yc, xc, wx * wy, valid.astype(jnp.float32)))
    return out


def _gather2d(img, yc, xc, H, W):
    M = yc.shape[0]
    yhot = (jax.lax.broadcasted_iota(jnp.int32, (M, H), 1)
            == yc[:, None]).astype(jnp.float32)
    rows = jax.lax.dot_general(yhot, img, (((1,), (0,)), ((), ())),
                               preferred_element_type=jnp.float32,
                               precision=_HIGH)
    xmask = (jax.lax.broadcasted_iota(jnp.int32, (M, W), 1)
             == xc[:, None]).astype(jnp.float32)
    return jnp.sum(xmask * rows, axis=1)


def _interp(img, corner_data):
    acc = None
    H, W = img.shape
    for yc, xc, w, valid in corner_data:
        g = _gather2d(img, yc, xc, H, W)
        t = (g * valid) * w
        acc = t if acc is None else acc + t
    return acc


def _head_kernel(out_ref, proj_ref, over_ref, cov_ref, wc_ref, b_ref,
                 rend_ref, pts_ref):
    B, _, Ho, Wo = out_ref.shape
    Hf, Wf = proj_ref.shape[2], proj_ref.shape[3]
    P = over_ref.shape[2]
    nb = pts_ref.shape[2] - cov_ref.shape[2]
    for bi in range(B):
        ox = over_ref[bi, 0]
        oy = over_ref[bi, 1]
        ch0 = out_ref[bi, 0]
        ch1 = out_ref[bi, 1]
        cd = _corners(ox, oy, Ho, Wo)
        a = _interp(ch0, cd)
        b2 = _interp(ch1, cd)
        u = -1.0 * (jnp.maximum(a, b2) - jnp.minimum(a, b2))
        gt = (u[None, :] > u[:, None]).astype(jnp.float32)
        eq = (u[None, :] == u[:, None])
        jlt = (jax.lax.broadcasted_iota(jnp.int32, (P, P), 1)
               < jax.lax.broadcasted_iota(jnp.int32, (P, P), 0))
        rank = jnp.sum(gt + (eq & jlt).astype(jnp.float32),
                       axis=1).astype(jnp.int32)
        sel = (jax.lax.broadcasted_iota(jnp.int32, (nb, P), 0)
               == rank[None, :]).astype(jnp.float32)
        px_i = jnp.sum(sel * ox[None, :], axis=1)
        py_i = jnp.sum(sel * oy[None, :], axis=1)
        pts_ref[bi, 0, :nb] = px_i
        pts_ref[bi, 1, :nb] = py_i
        pts_ref[bi, 0, nb:] = cov_ref[bi, 0]
        pts_ref[bi, 1, nb:] = cov_ref[bi, 1]
        px = jnp.concatenate([px_i, cov_ref[bi, 0]], axis=0)
        py = jnp.concatenate([py_i, cov_ref[bi, 1]], axis=0)
        cdo = _corners(px, py, Ho, Wo)
        coarse0 = _interp(ch0, cdo)
        coarse1 = _interp(ch1, cdo)
        cdf = _corners(px, py, Hf, Wf)
        fine0 = _interp(proj_ref[bi, 0], cdf)
        fine1 = _interp(proj_ref[bi, 1], cdf)
        for oi, fine in ((0, fine0), (1, fine1)):
            rend_ref[bi, oi, :] = (wc_ref[oi, 0] * coarse0
                                   + wc_ref[oi, 1] * coarse1
                                   + fine + b_ref[oi])


def kernel(x, res2, out, W, b):
    B, C, Hf, Wf = res2.shape
    N = x.shape[-1] // 16
    nb = int(_BETA * N)
    rng = jax.random.key(42)
    r1, r2 = jax.random.split(rng)
    over = jax.random.uniform(r1, (B, _K * N, 2), dtype=jnp.float32)
    coverage = jax.random.uniform(r2, (B, N - nb, 2), dtype=jnp.float32)
    over_t = jnp.transpose(over, (0, 2, 1))
    cov_t = jnp.transpose(coverage, (0, 2, 1))

    HW = Hf * Wf
    blk = 2048
    proj = pl.pallas_call(
        _proj_kernel,
        grid=(B, HW // blk),
        in_specs=[
            pl.BlockSpec((1, C, blk), lambda bi, j: (bi, 0, j)),
            pl.BlockSpec((W.shape[0], C), lambda bi, j: (0, 0)),
        ],
        out_specs=pl.BlockSpec((1, W.shape[0], blk), lambda bi, j: (bi, 0, j)),
        out_shape=jax.ShapeDtypeStruct((B, W.shape[0], HW), jnp.float32),
    )(res2.reshape(B, C, HW), W[:, 2:])

    rend, pts_t = pl.pallas_call(
        _head_kernel,
        in_specs=[
            pl.BlockSpec(out.shape, lambda: (0, 0, 0, 0)),
            pl.BlockSpec((B, W.shape[0], Hf, Wf), lambda: (0, 0, 0, 0)),
            pl.BlockSpec(over_t.shape, lambda: (0, 0, 0)),
            pl.BlockSpec(cov_t.shape, lambda: (0, 0, 0)),
            pl.BlockSpec(memory_space=pltpu.MemorySpace.SMEM),
            pl.BlockSpec(memory_space=pltpu.MemorySpace.SMEM),
        ],
        out_specs=[
            pl.BlockSpec((B, W.shape[0], N), lambda: (0, 0, 0)),
            pl.BlockSpec((B, 2, N), lambda: (0, 0, 0)),
        ],
        out_shape=[
            jax.ShapeDtypeStruct((B, W.shape[0], N), jnp.float32),
            jax.ShapeDtypeStruct((B, 2, N), jnp.float32),
        ],
    )(out, proj.reshape(B, W.shape[0], Hf, Wf), over_t, cov_t, W[:, :2], b)

    return rend, jnp.transpose(pts_t, (0, 2, 1))


# speedup vs baseline: 1.8749x; 1.8749x over previous
import functools

import jax
import jax.numpy as jnp
from jax import lax
from jax.experimental import pallas as pl
from jax.experimental.pallas import tpu as pltpu
from jax.experimental.pallas import tpu_sc as plsc

_K = 3
_BETA = 0.75
_HIGH = jax.lax.Precision.HIGHEST

_NW = 32
_ROWS_PER_W = 2048 // _NW


def _corners(px, py, H, W):
    gx = 2.0 * px - 1.0
    gy = 2.0 * py - 1.0
    fx = ((gx + 1.0) * W - 1.0) / 2.0
    fy = ((gy + 1.0) * H - 1.0) / 2.0
    x0 = jnp.floor(fx)
    y0 = jnp.floor(fy)
    x1 = x0 + 1.0
    y1 = y0 + 1.0
    wx1 = fx - x0
    wx0 = 1.0 - wx1
    wy1 = fy - y0
    wy0 = 1.0 - wy1
    out = []
    for xi, yi, wx, wy in ((x0, y0, wx0, wy0), (x1, y0, wx1, wy0),
                           (x0, y1, wx0, wy1), (x1, y1, wx1, wy1)):
        valid = ((xi >= 0) & (xi <= W - 1) & (yi >= 0) & (yi <= H - 1))
        xc = jnp.clip(xi, 0, W - 1).astype(jnp.int32)
        yc = jnp.clip(yi, 0, H - 1).astype(jnp.int32)
        out.append((yc, xc, wx * wy, valid.astype(jnp.float32)))
    return out


def _gather2d(img, yc, xc, H, W):
    M = yc.shape[0]
    yhot = (jax.lax.broadcasted_iota(jnp.int32, (M, H), 1)
            == yc[:, None]).astype(jnp.float32)
    rows = jax.lax.dot_general(yhot, img, (((1,), (0,)), ((), ())),
                               preferred_element_type=jnp.float32,
                               precision=_HIGH)
    xmask = (jax.lax.broadcasted_iota(jnp.int32, (M, W), 1)
             == xc[:, None]).astype(jnp.float32)
    return jnp.sum(xmask * rows, axis=1)


def _interp(img, corner_data):
    acc = None
    H, W = img.shape
    for yc, xc, w, valid in corner_data:
        g = _gather2d(img, yc, xc, H, W)
        t = (g * valid) * w
        acc = t if acc is None else acc + t
    return acc


def _head_kernel(out_ref, over_ref, cov_ref, wc_ref, b_ref,
                 rc_ref, pts_ref, idx_ref, w4_ref):
    B, _, Ho, Wo = out_ref.shape
    P = over_ref.shape[2]
    N = pts_ref.shape[2]
    nb = N - cov_ref.shape[2]
    pxs, pys = [], []
    for bi in range(B):
        ox = over_ref[bi, 0]
        oy = over_ref[bi, 1]
        ch0 = out_ref[bi, 0]
        ch1 = out_ref[bi, 1]
        cd = _corners(ox, oy, Ho, Wo)
        a = _interp(ch0, cd)
        b2 = _interp(ch1, cd)
        u = -1.0 * (jnp.maximum(a, b2) - jnp.minimum(a, b2))
        gt = (u[None, :] > u[:, None]).astype(jnp.float32)
        eq = (u[None, :] == u[:, None])
        jlt = (jax.lax.broadcasted_iota(jnp.int32, (P, P), 1)
               < jax.lax.broadcasted_iota(jnp.int32, (P, P), 0))
        rank = jnp.sum(gt + (eq & jlt).astype(jnp.float32),
                       axis=1).astype(jnp.int32)
        sel = (jax.lax.broadcasted_iota(jnp.int32, (nb, P), 0)
               == rank[None, :]).astype(jnp.float32)
        px_i = jnp.sum(sel * ox[None, :], axis=1)
        py_i = jnp.sum(sel * oy[None, :], axis=1)
        pts_ref[bi, 0, :nb] = px_i
        pts_ref[bi, 1, :nb] = py_i
        pts_ref[bi, 0, nb:] = cov_ref[bi, 0]
        pts_ref[bi, 1, nb:] = cov_ref[bi, 1]
        px = jnp.concatenate([px_i, cov_ref[bi, 0]], axis=0)
        py = jnp.concatenate([py_i, cov_ref[bi, 1]], axis=0)
        pxs.append(px)
        pys.append(py)
        cdo = _corners(px, py, Ho, Wo)
        coarse0 = _interp(ch0, cdo)
        coarse1 = _interp(ch1, cdo)
        for oi in range(2):
            rc_ref[bi, oi, :] = (wc_ref[oi, 0] * coarse0
                                 + wc_ref[oi, 1] * coarse1 + b_ref[oi])
    px_all = jnp.concatenate(pxs, axis=0)
    py_all = jnp.concatenate(pys, axis=0)
    Hf, Wf, C = 128, 256, 512
    boff = jnp.where(
        jax.lax.broadcasted_iota(jnp.int32, (1, B * N), 1) >= N,
        C * Hf * Wf, 0)[0]
    cdf = _corners(px_all, py_all, Hf, Wf)
    bases, ws = [], []
    for yc, xc, w, valid in cdf:
        bases.append(boff + yc * Wf + xc)
        ws.append(w * valid)
    base_all = jnp.concatenate(bases, axis=0)
    for j in range(4):
        w4_ref[j, :] = ws[j]
    idx_ref[...] = (base_all[None, :]
                    + jax.lax.broadcasted_iota(jnp.int32, (C, 4 * B * N), 0)
                    * (Hf * Wf))


def _fine_kernel(g_ref, wf_ref, w4_ref, rc_ref, rend_ref):
    B = rend_ref.shape[0]
    N = rend_ref.shape[2]
    d = jax.lax.dot_general(wf_ref[...], g_ref[...], (((1,), (0,)), ((), ())),
                            preferred_element_type=jnp.float32,
                            precision=_HIGH)
    fine = None
    for j in range(4):
        t = w4_ref[j][None, :] * d[:, j * B * N:(j + 1) * B * N]
        fine = t if fine is None else fine + t
    for bi in range(B):
        for oi in range(2):
            rend_ref[bi, oi, :] = (rc_ref[bi, oi]
                                   + fine[oi, bi * N:(bi + 1) * N])


def _make_sc_gather(n_elems):
    mesh = plsc.VectorSubcoreMesh(core_axis_name="c", subcore_axis_name="s")

    @functools.partial(
        pl.kernel, mesh=mesh,
        out_type=jax.ShapeDtypeStruct((2048, 128), jnp.float32),
        scratch_types=[
            pltpu.VMEM((_ROWS_PER_W, 128), jnp.int32),
            pltpu.VMEM((_ROWS_PER_W, 128), jnp.float32),
            pltpu.SemaphoreType.DMA,
        ],
    )
    def sc_gather(res2_hbm, idx_hbm, g_hbm, idx_v, g_v, sem):
        wid = lax.axis_index("s") * 2 + lax.axis_index("c")
        base = wid * _ROWS_PER_W
        pltpu.sync_copy(idx_hbm.at[pl.ds(base, _ROWS_PER_W)], idx_v)

        def fire(r, carry):
            pltpu.async_copy(res2_hbm.at[idx_v.at[r]], g_v.at[r], sem)
            return carry

        lax.fori_loop(0, _ROWS_PER_W, fire, 0)

        def drain(r, carry):
            pltpu.make_async_copy(res2_hbm.at[idx_v.at[r]], g_v.at[r],
                                  sem).wait()
            return carry

        lax.fori_loop(0, _ROWS_PER_W, drain, 0)
        pltpu.sync_copy(g_v, g_hbm.at[pl.ds(base, _ROWS_PER_W)])

    return sc_gather


def kernel(x, res2, out, W, b):
    B, C, Hf, Wf = res2.shape
    N = x.shape[-1] // 16
    nb = int(_BETA * N)
    rng = jax.random.key(42)
    r1, r2 = jax.random.split(rng)
    over = jax.random.uniform(r1, (B, _K * N, 2), dtype=jnp.float32)
    coverage = jax.random.uniform(r2, (B, N - nb, 2), dtype=jnp.float32)
    over_t = jnp.transpose(over, (0, 2, 1))
    cov_t = jnp.transpose(coverage, (0, 2, 1))

    rc, pts_t, idx, w4 = pl.pallas_call(
        _head_kernel,
        in_specs=[
            pl.BlockSpec(out.shape, lambda: (0, 0, 0, 0)),
            pl.BlockSpec(over_t.shape, lambda: (0, 0, 0)),
            pl.BlockSpec(cov_t.shape, lambda: (0, 0, 0)),
            pl.BlockSpec(memory_space=pltpu.MemorySpace.SMEM),
            pl.BlockSpec(memory_space=pltpu.MemorySpace.SMEM),
        ],
        out_specs=[
            pl.BlockSpec((B, 2, N), lambda: (0, 0, 0)),
            pl.BlockSpec((B, 2, N), lambda: (0, 0, 0)),
            pl.BlockSpec((C, 4 * B * N), lambda: (0, 0)),
            pl.BlockSpec((4, B * N), lambda: (0, 0)),
        ],
        out_shape=[
            jax.ShapeDtypeStruct((B, 2, N), jnp.float32),
            jax.ShapeDtypeStruct((B, 2, N), jnp.float32),
            jax.ShapeDtypeStruct((C, 4 * B * N), jnp.int32),
            jax.ShapeDtypeStruct((4, B * N), jnp.float32),
        ],
    )(out, over_t, cov_t, W[:, :2], b)

    n_elems = B * C * Hf * Wf
    g = _make_sc_gather(n_elems)(res2.reshape(n_elems),
                                 idx.reshape(2048, 128))

    rend = pl.pallas_call(
        _fine_kernel,
        in_specs=[
            pl.BlockSpec((C, 4 * B * N), lambda: (0, 0)),
            pl.BlockSpec((2, C), lambda: (0, 0)),
            pl.BlockSpec((4, B * N), lambda: (0, 0)),
            pl.BlockSpec((B, 2, N), lambda: (0, 0, 0)),
        ],
        out_specs=pl.BlockSpec((B, 2, N), lambda: (0, 0, 0)),
        out_shape=jax.ShapeDtypeStruct((B, 2, N), jnp.float32),
    )(g.reshape(C, 4 * B * N), W[:, 2:], w4, rc)

    return rend, jnp.transpose(pts_t, (0, 2, 1))
